# baseline (speedup 1.0000x reference)
import functools

import jax
import jax.numpy as jnp
from jax import lax
from jax.experimental import pallas as pl
from jax.experimental.pallas import tpu as pltpu
from jax.experimental.pallas import tpu_sc as plsc

N = 10000
E = 320000
D = 128
EPS = 1e-6

NC = 2
NS = 16
NW = NC * NS
EW = E // NW
CH = 128
NFULL = EW // CH
TAIL = EW - NFULL * CH
N_PAD = 10240
RPT = N_PAD // NS

_MESH = dict(core_axis_name="c", subcore_axis_name="s")


def _zero_rows(buf, nrows):
    def fill(i, _):
        for j in range(D // 16):
            buf[i, pl.ds(j * 16, 16)] = jnp.zeros((16,), jnp.float32)
        return 0

    lax.fori_loop(0, nrows, fill, 0)


def _agg_body(h, src, dst, agg_out, src_v, dst_v, src_t, dst_t, rows0,
              rows1, rows_t, acc_sh, ss0, sd0, gs0, cs0, ss1, sd1, gs1, cs1,
              st):
    cid = lax.axis_index("c")
    sid = lax.axis_index("s")
    wid = sid * NC + cid
    r0 = sid * RPT

    _zero_rows(rows0, CH)

    def zspm(j, _):
        pltpu.sync_copy(rows0, acc_sh.at[pl.ds(r0 + j * CH, CH)])
        return 0

    lax.fori_loop(0, RPT // CH, zspm, 0)
    plsc.subcore_barrier()

    ebase = wid * EW

    def issue_loads(g, b, ssem, dsem):
        off = ebase + g * CH
        pltpu.async_copy(src.at[pl.ds(off, CH)], src_v.at[b], ssem)
        pltpu.async_copy(dst.at[pl.ds(off, CH)], dst_v.at[b], dsem)

    def start_gather(b, rows, ssem, gsem):
        pltpu.make_async_copy(src.at[pl.ds(0, CH)], src_v.at[b], ssem).wait()
        pltpu.async_copy(h.at[src_v.at[b]], rows, gsem)

    def start_scatter(b, rows, gsem, dsem, csem):
        pltpu.make_async_copy(h.at[src_v.at[b]], rows, gsem).wait()
        pltpu.make_async_copy(dst.at[pl.ds(0, CH)], dst_v.at[b], dsem).wait()
        pltpu.async_copy(rows, acc_sh.at[dst_v.at[b]], csem, add=True)

    def wait_scatter(b, rows, csem):
        pltpu.make_async_copy(rows, acc_sh.at[dst_v.at[b]], csem).wait()

    issue_loads(0, 0, ss0, sd0)
    issue_loads(1, 1, ss1, sd1)
    start_gather(0, rows0, ss0, gs0)
    start_gather(1, rows1, ss1, gs1)

    def pipe(i, _):
        g0 = 2 * i
        start_scatter(0, rows0, gs0, sd0, cs0)
        start_scatter(1, rows1, gs1, sd1, cs1)
        wait_scatter(0, rows0, cs0)
        issue_loads(g0 + 2, 0, ss0, sd0)
        start_gather(0, rows0, ss0, gs0)
        wait_scatter(1, rows1, cs1)
        issue_loads(g0 + 3, 1, ss1, sd1)
        start_gather(1, rows1, ss1, gs1)
        return 0

    lax.fori_loop(0, NFULL // 2 - 1, pipe, 0)
    start_scatter(0, rows0, gs0, sd0, cs0)
    start_scatter(1, rows1, gs1, sd1, cs1)
    wait_scatter(0, rows0, cs0)
    wait_scatter(1, rows1, cs1)

    offt = ebase + NFULL * CH
    pltpu.sync_copy(src.at[pl.ds(offt, TAIL)], src_t)
    pltpu.sync_copy(dst.at[pl.ds(offt, TAIL)], dst_t.at[0])
    pltpu.async_copy(h.at[src_t], rows_t, st).wait()
    pltpu.sync_copy(rows_t, acc_sh.at[dst_t.at[0]], add=True)
    plsc.subcore_barrier()

    def cout(j, _):
        rr = r0 + j * CH
        pltpu.sync_copy(acc_sh.at[pl.ds(rr, CH)], rows0)
        pltpu.sync_copy(rows0, agg_out.at[cid, pl.ds(rr, CH)])
        return 0

    lax.fori_loop(0, RPT // CH, cout, 0)


def _deg_body(dst, deg_out, dst_v, dst_t, ones_v, acc_sh, sd0, sd1, cs0,
              cs1):
    cid = lax.axis_index("c")
    sid = lax.axis_index("s")
    wid = sid * NC + cid
    r0 = sid * RPT

    _zero_rows(ones_v, CH)

    def zspm(j, _):
        pltpu.sync_copy(ones_v, acc_sh.at[pl.ds(r0 + j * CH, CH)])
        return 0

    lax.fori_loop(0, RPT // CH, zspm, 0)

    def fill1(i, _):
        for j in range(D // 16):
            ones_v[i, pl.ds(j * 16, 16)] = jnp.ones((16,), jnp.float32)
        return 0

    lax.fori_loop(0, CH, fill1, 0)
    plsc.subcore_barrier()

    ebase = wid * EW

    def load_dst(g, b, dsem):
        pltpu.async_copy(dst.at[pl.ds(ebase + g * CH, CH)], dst_v.at[b], dsem)

    def start_scatter(b, dsem, csem):
        pltpu.make_async_copy(dst.at[pl.ds(0, CH)], dst_v.at[b], dsem).wait()
        pltpu.async_copy(ones_v, acc_sh.at[dst_v.at[b]], csem, add=True)

    def wait_scatter(b, csem):
        pltpu.make_async_copy(ones_v, acc_sh.at[dst_v.at[b]], csem).wait()

    load_dst(0, 0, sd0)
    load_dst(1, 1, sd1)

    def pipe(i, _):
        g0 = 2 * i
        start_scatter(0, sd0, cs0)
        start_scatter(1, sd1, cs1)
        wait_scatter(0, cs0)
        load_dst(g0 + 2, 0, sd0)
        wait_scatter(1, cs1)
        load_dst(g0 + 3, 1, sd1)
        return 0

    lax.fori_loop(0, NFULL // 2 - 1, pipe, 0)
    start_scatter(0, sd0, cs0)
    start_scatter(1, sd1, cs1)
    wait_scatter(0, cs0)
    wait_scatter(1, cs1)

    offt = ebase + NFULL * CH
    pltpu.sync_copy(dst.at[pl.ds(offt, TAIL)], dst_t.at[0])
    pltpu.sync_copy(ones_v.at[pl.ds(0, TAIL)], acc_sh.at[dst_t.at[0]],
                    add=True)
    plsc.subcore_barrier()

    def cout(j, _):
        rr = r0 + j * CH
        pltpu.sync_copy(acc_sh.at[pl.ds(rr, CH)], ones_v)
        pltpu.sync_copy(ones_v, deg_out.at[cid, pl.ds(rr, CH)])
        return 0

    lax.fori_loop(0, RPT // CH, cout, 0)


def _make_agg():
    return functools.partial(
        pl.kernel,
        mesh=plsc.VectorSubcoreMesh(**_MESH),
        out_type=jax.ShapeDtypeStruct((NC, N_PAD, D), jnp.float32),
        scratch_types=[
            pltpu.VMEM((2, CH), jnp.int32),
            pltpu.VMEM((2, CH), jnp.int32),
            pltpu.VMEM((TAIL,), jnp.int32),
            pltpu.VMEM((1, TAIL), jnp.int32),
            pltpu.VMEM((CH, D), jnp.float32),
            pltpu.VMEM((CH, D), jnp.float32),
            pltpu.VMEM((TAIL, D), jnp.float32),
            pltpu.VMEM_SHARED((N_PAD, D), jnp.float32),
            pltpu.SemaphoreType.DMA,
            pltpu.SemaphoreType.DMA,
            pltpu.SemaphoreType.DMA,
            pltpu.SemaphoreType.DMA,
            pltpu.SemaphoreType.DMA,
            pltpu.SemaphoreType.DMA,
            pltpu.SemaphoreType.DMA,
            pltpu.SemaphoreType.DMA,
            pltpu.SemaphoreType.DMA,
        ],
    )(_agg_body)


def _make_deg():
    return functools.partial(
        pl.kernel,
        mesh=plsc.VectorSubcoreMesh(**_MESH),
        out_type=jax.ShapeDtypeStruct((NC, N_PAD, D), jnp.float32),
        scratch_types=[
            pltpu.VMEM((2, CH), jnp.int32),
            pltpu.VMEM((1, TAIL), jnp.int32),
            pltpu.VMEM((CH, D), jnp.float32),
            pltpu.VMEM_SHARED((N_PAD, D), jnp.float32),
            pltpu.SemaphoreType.DMA,
            pltpu.SemaphoreType.DMA,
            pltpu.SemaphoreType.DMA,
            pltpu.SemaphoreType.DMA,
        ],
    )(_deg_body)


def _logmap_body(x_ref, o_ref):
    v = x_ref[...]
    n = jnp.sqrt(jnp.sum(v * v, axis=1, keepdims=True))
    nc = jnp.clip(n, EPS, 1.0 - 1e-5)
    o_ref[...] = (0.5 * jnp.log((1.0 + nc) / (1.0 - nc))) * v / nc


def _tc_logmap(x):
    blk = 1000
    return pl.pallas_call(
        _logmap_body,
        out_shape=jax.ShapeDtypeStruct((N, D), jnp.float32),
        grid=(N // blk,),
        in_specs=[pl.BlockSpec((blk, D), lambda i: (i, 0))],
        out_specs=pl.BlockSpec((blk, D), lambda i: (i, 0)),
    )(x)


def _combine_body(agg_ref, deg_ref, w_ref, o_ref, *, last):
    a = agg_ref[0] + agg_ref[1]
    d = jnp.sum(deg_ref[...], axis=(0, 2)) * (1.0 / D)
    a = a / jnp.clip(d, 1.0, None)[:, None]
    out = jnp.dot(a, w_ref[...], preferred_element_type=jnp.float32)
    n = jnp.sqrt(jnp.sum(out * out, axis=1, keepdims=True))
    nc = jnp.clip(n, EPS, None)
    y = jnp.tanh(nc) * out / nc
    if not last:
        m = jnp.sqrt(jnp.sum(y * y, axis=1, keepdims=True))
        mc = jnp.clip(m, EPS, 1.0 - 1e-5)
        y = (0.5 * jnp.log((1.0 + mc) / (1.0 - mc))) * y / mc
    o_ref[...] = y


def _tc_combine(agg, deg, w, last):
    blk = 1024
    return pl.pallas_call(
        functools.partial(_combine_body, last=last),
        out_shape=jax.ShapeDtypeStruct((N_PAD, D), jnp.float32),
        grid=(N_PAD // blk,),
        in_specs=[
            pl.BlockSpec((NC, blk, D), lambda i: (0, i, 0)),
            pl.BlockSpec((NC, blk, D), lambda i: (0, i, 0)),
            pl.BlockSpec((D, D), lambda i: (0, 0)),
        ],
        out_specs=pl.BlockSpec((blk, D), lambda i: (i, 0)),
    )(agg, deg, w)


def kernel(x, edge_index, W0, W1):
    src = edge_index[0].astype(jnp.int32)
    dst = edge_index[1].astype(jnp.int32)
    h0 = _tc_logmap(x)
    deg = _make_deg()(dst)
    agg1 = _make_agg()(h0, src, dst)
    h1 = _tc_combine(agg1, deg, W0, last=False)
    agg2 = _make_agg()(h1, src, dst)
    y = _tc_combine(agg2, deg, W1, last=True)
    return y[:N]


# speedup vs baseline: 1.1206x; 1.1206x over previous
import functools

import jax
import jax.numpy as jnp
from jax import lax
from jax.experimental import pallas as pl
from jax.experimental.pallas import tpu as pltpu
from jax.experimental.pallas import tpu_sc as plsc

N = 10000
E = 320000
D = 128
EPS = 1e-6

NC = 2
NS = 16
NW = NC * NS
EW = E // NW
CH = 128
NFULL = EW // CH
TAIL = EW - NFULL * CH
N_PAD = 10240
RPT = N_PAD // NS

_MESH = dict(core_axis_name="c", subcore_axis_name="s")


def _zero_rows(buf, nrows):
    def fill(i, _):
        for j in range(D // 16):
            buf[i, pl.ds(j * 16, 16)] = jnp.zeros((16,), jnp.float32)
        return 0

    lax.fori_loop(0, nrows, fill, 0)


def _agg_body(h, src, dst, agg_out, src_v, dst_v, src_t, dst_t, rows0,
              rows1, rows_t, acc_sh, ss0, sd0, gs0, cs0, ss1, sd1, gs1, cs1,
              st):
    cid = lax.axis_index("c")
    sid = lax.axis_index("s")
    wid = sid * NC + cid
    r0 = sid * RPT

    _zero_rows(rows0, CH)

    def zspm(j, _):
        pltpu.sync_copy(rows0, acc_sh.at[pl.ds(r0 + j * CH, CH)])
        return 0

    lax.fori_loop(0, RPT // CH, zspm, 0)
    plsc.subcore_barrier()

    ebase = wid * EW

    def issue_loads(g, b, ssem, dsem):
        off = ebase + g * CH
        pltpu.async_copy(src.at[pl.ds(off, CH)], src_v.at[b], ssem)
        pltpu.async_copy(dst.at[pl.ds(off, CH)], dst_v.at[b], dsem)

    def start_gather(b, rows, ssem, gsem):
        pltpu.make_async_copy(src.at[pl.ds(0, CH)], src_v.at[b], ssem).wait()
        pltpu.async_copy(h.at[src_v.at[b]], rows, gsem)

    def finish_scatter(b, rows, gsem, dsem):
        pltpu.make_async_copy(h.at[src_v.at[b]], rows, gsem).wait()
        pltpu.make_async_copy(dst.at[pl.ds(0, CH)], dst_v.at[b], dsem).wait()
        pltpu.sync_copy(rows, acc_sh.at[dst_v.at[b]], add=True)

    issue_loads(0, 0, ss0, sd0)
    issue_loads(1, 1, ss1, sd1)
    start_gather(0, rows0, ss0, gs0)
    start_gather(1, rows1, ss1, gs1)

    def pipe(i, _):
        g0 = 2 * i
        finish_scatter(0, rows0, gs0, sd0)
        issue_loads(g0 + 2, 0, ss0, sd0)
        start_gather(0, rows0, ss0, gs0)
        finish_scatter(1, rows1, gs1, sd1)
        issue_loads(g0 + 3, 1, ss1, sd1)
        start_gather(1, rows1, ss1, gs1)
        return 0

    lax.fori_loop(0, NFULL // 2 - 1, pipe, 0)
    finish_scatter(0, rows0, gs0, sd0)
    finish_scatter(1, rows1, gs1, sd1)

    offt = ebase + NFULL * CH
    pltpu.sync_copy(src.at[pl.ds(offt, TAIL)], src_t)
    pltpu.sync_copy(dst.at[pl.ds(offt, TAIL)], dst_t.at[0])
    pltpu.async_copy(h.at[src_t], rows_t, st).wait()
    pltpu.sync_copy(rows_t, acc_sh.at[dst_t.at[0]], add=True)
    plsc.subcore_barrier()

    pltpu.sync_copy(acc_sh.at[pl.ds(r0, RPT)], agg_out.at[cid, pl.ds(r0, RPT)])


def _deg_body(dst, deg_out, dst_v, dst_t, ones_v, acc_sh, sd0, sd1, cs0,
              cs1):
    cid = lax.axis_index("c")
    sid = lax.axis_index("s")
    wid = sid * NC + cid
    r0 = sid * RPT

    _zero_rows(ones_v, CH)

    def zspm(j, _):
        pltpu.sync_copy(ones_v, acc_sh.at[pl.ds(r0 + j * CH, CH)])
        return 0

    lax.fori_loop(0, RPT // CH, zspm, 0)

    def fill1(i, _):
        for j in range(D // 16):
            ones_v[i, pl.ds(j * 16, 16)] = jnp.ones((16,), jnp.float32)
        return 0

    lax.fori_loop(0, CH, fill1, 0)
    plsc.subcore_barrier()

    ebase = wid * EW

    def load_dst(g, b, dsem):
        pltpu.async_copy(dst.at[pl.ds(ebase + g * CH, CH)], dst_v.at[b], dsem)

    def do_scatter(b, dsem):
        pltpu.make_async_copy(dst.at[pl.ds(0, CH)], dst_v.at[b], dsem).wait()
        pltpu.sync_copy(ones_v, acc_sh.at[dst_v.at[b]], add=True)

    load_dst(0, 0, sd0)
    load_dst(1, 1, sd1)

    def pipe(i, _):
        g0 = 2 * i
        do_scatter(0, sd0)
        load_dst(g0 + 2, 0, sd0)
        do_scatter(1, sd1)
        load_dst(g0 + 3, 1, sd1)
        return 0

    lax.fori_loop(0, NFULL // 2 - 1, pipe, 0)
    do_scatter(0, sd0)
    do_scatter(1, sd1)

    offt = ebase + NFULL * CH
    pltpu.sync_copy(dst.at[pl.ds(offt, TAIL)], dst_t.at[0])
    pltpu.sync_copy(ones_v.at[pl.ds(0, TAIL)], acc_sh.at[dst_t.at[0]],
                    add=True)
    plsc.subcore_barrier()

    pltpu.sync_copy(acc_sh.at[pl.ds(r0, RPT)], deg_out.at[cid, pl.ds(r0, RPT)])


def _make_agg():
    return functools.partial(
        pl.kernel,
        mesh=plsc.VectorSubcoreMesh(**_MESH),
        out_type=jax.ShapeDtypeStruct((NC, N_PAD, D), jnp.float32),
        scratch_types=[
            pltpu.VMEM((2, CH), jnp.int32),
            pltpu.VMEM((2, CH), jnp.int32),
            pltpu.VMEM((TAIL,), jnp.int32),
            pltpu.VMEM((1, TAIL), jnp.int32),
            pltpu.VMEM((CH, D), jnp.float32),
            pltpu.VMEM((CH, D), jnp.float32),
            pltpu.VMEM((TAIL, D), jnp.float32),
            pltpu.VMEM_SHARED((N_PAD, D), jnp.float32),
            pltpu.SemaphoreType.DMA,
            pltpu.SemaphoreType.DMA,
            pltpu.SemaphoreType.DMA,
            pltpu.SemaphoreType.DMA,
            pltpu.SemaphoreType.DMA,
            pltpu.SemaphoreType.DMA,
            pltpu.SemaphoreType.DMA,
            pltpu.SemaphoreType.DMA,
            pltpu.SemaphoreType.DMA,
        ],
    )(_agg_body)


def _make_deg():
    return functools.partial(
        pl.kernel,
        mesh=plsc.VectorSubcoreMesh(**_MESH),
        out_type=jax.ShapeDtypeStruct((NC, N_PAD, D), jnp.float32),
        scratch_types=[
            pltpu.VMEM((2, CH), jnp.int32),
            pltpu.VMEM((1, TAIL), jnp.int32),
            pltpu.VMEM((CH, D), jnp.float32),
            pltpu.VMEM_SHARED((N_PAD, D), jnp.float32),
            pltpu.SemaphoreType.DMA,
            pltpu.SemaphoreType.DMA,
            pltpu.SemaphoreType.DMA,
            pltpu.SemaphoreType.DMA,
        ],
    )(_deg_body)


def _logmap_body(x_ref, o_ref):
    v = x_ref[...]
    n = jnp.sqrt(jnp.sum(v * v, axis=1, keepdims=True))
    nc = jnp.clip(n, EPS, 1.0 - 1e-5)
    o_ref[...] = (0.5 * jnp.log((1.0 + nc) / (1.0 - nc))) * v / nc


def _tc_logmap(x):
    blk = 1000
    return pl.pallas_call(
        _logmap_body,
        out_shape=jax.ShapeDtypeStruct((N, D), jnp.float32),
        grid=(N // blk,),
        in_specs=[pl.BlockSpec((blk, D), lambda i: (i, 0))],
        out_specs=pl.BlockSpec((blk, D), lambda i: (i, 0)),
    )(x)


def _combine_body(agg_ref, deg_ref, w_ref, o_ref, *, last):
    a = agg_ref[0] + agg_ref[1]
    d = jnp.sum(deg_ref[...], axis=(0, 2)) * (1.0 / D)
    a = a / jnp.clip(d, 1.0, None)[:, None]
    out = jnp.dot(a, w_ref[...], preferred_element_type=jnp.float32)
    n = jnp.sqrt(jnp.sum(out * out, axis=1, keepdims=True))
    nc = jnp.clip(n, EPS, None)
    y = jnp.tanh(nc) * out / nc
    if not last:
        m = jnp.sqrt(jnp.sum(y * y, axis=1, keepdims=True))
        mc = jnp.clip(m, EPS, 1.0 - 1e-5)
        y = (0.5 * jnp.log((1.0 + mc) / (1.0 - mc))) * y / mc
    o_ref[...] = y


def _tc_combine(agg, deg, w, last):
    blk = 1024
    return pl.pallas_call(
        functools.partial(_combine_body, last=last),
        out_shape=jax.ShapeDtypeStruct((N_PAD, D), jnp.float32),
        grid=(N_PAD // blk,),
        in_specs=[
            pl.BlockSpec((NC, blk, D), lambda i: (0, i, 0)),
            pl.BlockSpec((NC, blk, D), lambda i: (0, i, 0)),
            pl.BlockSpec((D, D), lambda i: (0, 0)),
        ],
        out_specs=pl.BlockSpec((blk, D), lambda i: (i, 0)),
    )(agg, deg, w)


def kernel(x, edge_index, W0, W1):
    src = edge_index[0].astype(jnp.int32)
    dst = edge_index[1].astype(jnp.int32)
    h0 = _tc_logmap(x)
    deg = _make_deg()(dst)
    agg1 = _make_agg()(h0, src, dst)
    h1 = _tc_combine(agg1, deg, W0, last=False)
    agg2 = _make_agg()(h1, src, dst)
    y = _tc_combine(agg2, deg, W1, last=True)
    return y[:N]


# speedup vs baseline: 1.1307x; 1.0090x over previous
import functools

import jax
import jax.numpy as jnp
from jax import lax
from jax.experimental import pallas as pl
from jax.experimental.pallas import tpu as pltpu
from jax.experimental.pallas import tpu_sc as plsc

N = 10000
E = 320000
D = 128
EPS = 1e-6

NC = 2
NS = 16
NW = NC * NS
EW = E // NW
CH = 128
NFULL = EW // CH
TAIL = EW - NFULL * CH
N_PAD = 10240
RPT = N_PAD // NS

_MESH = dict(core_axis_name="c", subcore_axis_name="s")


def _zero_rows(buf, nrows):
    def fill(i, _):
        for j in range(D // 16):
            buf[i, pl.ds(j * 16, 16)] = jnp.zeros((16,), jnp.float32)
        return 0

    lax.fori_loop(0, nrows, fill, 0)


def _agg_body(h, src, dst, agg_out, src_v, dst_v, src_t, dst_t, rows0,
              rows1, rows_t, acc_sh, ss0, sd0, gs0, ss1, sd1, gs1, st):
    cid = lax.axis_index("c")
    sid = lax.axis_index("s")
    wid = sid * NC + cid
    r0 = sid * RPT

    _zero_rows(rows0, CH)

    def zspm(j, _):
        pltpu.sync_copy(rows0, acc_sh.at[pl.ds(r0 + j * CH, CH)])
        return 0

    lax.fori_loop(0, RPT // CH, zspm, 0)
    plsc.subcore_barrier()

    ebase = wid * EW

    def issue_loads(g, b, ssem, dsem):
        off = ebase + g * CH
        pltpu.async_copy(src.at[pl.ds(off, CH)], src_v.at[b], ssem)
        pltpu.async_copy(dst.at[pl.ds(off, CH)], dst_v.at[b], dsem)

    def start_gather(b, rows, ssem, gsem):
        pltpu.make_async_copy(src.at[pl.ds(0, CH)], src_v.at[b], ssem).wait()
        pltpu.async_copy(h.at[src_v.at[b]], rows, gsem)

    def finish_scatter(b, rows, gsem, dsem):
        pltpu.make_async_copy(h.at[src_v.at[b]], rows, gsem).wait()
        pltpu.make_async_copy(dst.at[pl.ds(0, CH)], dst_v.at[b], dsem).wait()
        pltpu.sync_copy(rows, acc_sh.at[dst_v.at[b]], add=True)

    issue_loads(0, 0, ss0, sd0)
    issue_loads(1, 1, ss1, sd1)
    start_gather(0, rows0, ss0, gs0)
    start_gather(1, rows1, ss1, gs1)

    def pipe(i, _):
        g0 = 2 * i
        finish_scatter(0, rows0, gs0, sd0)
        issue_loads(g0 + 2, 0, ss0, sd0)
        start_gather(0, rows0, ss0, gs0)
        finish_scatter(1, rows1, gs1, sd1)
        issue_loads(g0 + 3, 1, ss1, sd1)
        start_gather(1, rows1, ss1, gs1)
        return 0

    lax.fori_loop(0, NFULL // 2 - 1, pipe, 0)
    finish_scatter(0, rows0, gs0, sd0)
    finish_scatter(1, rows1, gs1, sd1)

    offt = ebase + NFULL * CH
    pltpu.sync_copy(src.at[pl.ds(offt, TAIL)], src_t)
    pltpu.sync_copy(dst.at[pl.ds(offt, TAIL)], dst_t.at[0])
    pltpu.async_copy(h.at[src_t], rows_t, st).wait()
    pltpu.sync_copy(rows_t, acc_sh.at[dst_t.at[0]], add=True)
    plsc.subcore_barrier()

    pltpu.sync_copy(acc_sh.at[pl.ds(r0, RPT)], agg_out.at[cid, pl.ds(r0, RPT)])


def _agg_deg_body(h, src, dst, agg_out, deg_out, src_v, dst_v, src_t, dst_t,
                  rows0, rows1, rows_t, acc_sh, ss0, sd0, gs0, ss1, sd1, gs1,
                  st):
    cid = lax.axis_index("c")
    sid = lax.axis_index("s")
    wid = sid * NC + cid
    r0 = sid * RPT
    ebase = wid * EW
    offt = ebase + NFULL * CH

    _zero_rows(rows0, CH)

    def zspm(j, _):
        pltpu.sync_copy(rows0, acc_sh.at[pl.ds(r0 + j * CH, CH)])
        return 0

    lax.fori_loop(0, RPT // CH, zspm, 0)

    def fill1(i, _):
        for j in range(D // 16):
            rows0[i, pl.ds(j * 16, 16)] = jnp.ones((16,), jnp.float32)
        return 0

    lax.fori_loop(0, CH, fill1, 0)
    plsc.subcore_barrier()

    def load_dst(g, b, dsem):
        pltpu.async_copy(dst.at[pl.ds(ebase + g * CH, CH)], dst_v.at[b], dsem)

    def deg_scatter(b, dsem):
        pltpu.make_async_copy(dst.at[pl.ds(0, CH)], dst_v.at[b], dsem).wait()
        pltpu.sync_copy(rows0, acc_sh.at[dst_v.at[b]], add=True)

    load_dst(0, 0, sd0)
    load_dst(1, 1, sd1)

    def dpipe(i, _):
        g0 = 2 * i
        deg_scatter(0, sd0)
        load_dst(g0 + 2, 0, sd0)
        deg_scatter(1, sd1)
        load_dst(g0 + 3, 1, sd1)
        return 0

    lax.fori_loop(0, NFULL // 2 - 1, dpipe, 0)
    deg_scatter(0, sd0)
    deg_scatter(1, sd1)
    pltpu.sync_copy(dst.at[pl.ds(offt, TAIL)], dst_t.at[0])
    pltpu.sync_copy(rows0.at[pl.ds(0, TAIL)], acc_sh.at[dst_t.at[0]],
                    add=True)
    plsc.subcore_barrier()
    pltpu.sync_copy(acc_sh.at[pl.ds(r0, RPT)], deg_out.at[cid, pl.ds(r0, RPT)])

    _zero_rows(rows1, CH)

    def rzspm(j, _):
        pltpu.sync_copy(rows1, acc_sh.at[pl.ds(r0 + j * CH, CH)])
        return 0

    lax.fori_loop(0, RPT // CH, rzspm, 0)
    plsc.subcore_barrier()

    def issue_loads(g, b, ssem, dsem):
        off = ebase + g * CH
        pltpu.async_copy(src.at[pl.ds(off, CH)], src_v.at[b], ssem)
        pltpu.async_copy(dst.at[pl.ds(off, CH)], dst_v.at[b], dsem)

    def start_gather(b, rows, ssem, gsem):
        pltpu.make_async_copy(src.at[pl.ds(0, CH)], src_v.at[b], ssem).wait()
        pltpu.async_copy(h.at[src_v.at[b]], rows, gsem)

    def finish_scatter(b, rows, gsem, dsem):
        pltpu.make_async_copy(h.at[src_v.at[b]], rows, gsem).wait()
        pltpu.make_async_copy(dst.at[pl.ds(0, CH)], dst_v.at[b], dsem).wait()
        pltpu.sync_copy(rows, acc_sh.at[dst_v.at[b]], add=True)

    issue_loads(0, 0, ss0, sd0)
    issue_loads(1, 1, ss1, sd1)
    start_gather(0, rows0, ss0, gs0)
    start_gather(1, rows1, ss1, gs1)

    def pipe(i, _):
        g0 = 2 * i
        finish_scatter(0, rows0, gs0, sd0)
        issue_loads(g0 + 2, 0, ss0, sd0)
        start_gather(0, rows0, ss0, gs0)
        finish_scatter(1, rows1, gs1, sd1)
        issue_loads(g0 + 3, 1, ss1, sd1)
        start_gather(1, rows1, ss1, gs1)
        return 0

    lax.fori_loop(0, NFULL // 2 - 1, pipe, 0)
    finish_scatter(0, rows0, gs0, sd0)
    finish_scatter(1, rows1, gs1, sd1)

    pltpu.sync_copy(src.at[pl.ds(offt, TAIL)], src_t)
    pltpu.sync_copy(dst.at[pl.ds(offt, TAIL)], dst_t.at[0])
    pltpu.async_copy(h.at[src_t], rows_t, st).wait()
    pltpu.sync_copy(rows_t, acc_sh.at[dst_t.at[0]], add=True)
    plsc.subcore_barrier()
    pltpu.sync_copy(acc_sh.at[pl.ds(r0, RPT)], agg_out.at[cid, pl.ds(r0, RPT)])


def _make_agg():
    return functools.partial(
        pl.kernel,
        mesh=plsc.VectorSubcoreMesh(**_MESH),
        out_type=jax.ShapeDtypeStruct((NC, N_PAD, D), jnp.float32),
        scratch_types=[
            pltpu.VMEM((2, CH), jnp.int32),
            pltpu.VMEM((2, CH), jnp.int32),
            pltpu.VMEM((TAIL,), jnp.int32),
            pltpu.VMEM((1, TAIL), jnp.int32),
            pltpu.VMEM((CH, D), jnp.float32),
            pltpu.VMEM((CH, D), jnp.float32),
            pltpu.VMEM((TAIL, D), jnp.float32),
            pltpu.VMEM_SHARED((N_PAD, D), jnp.float32),
            pltpu.SemaphoreType.DMA,
            pltpu.SemaphoreType.DMA,
            pltpu.SemaphoreType.DMA,
            pltpu.SemaphoreType.DMA,
            pltpu.SemaphoreType.DMA,
            pltpu.SemaphoreType.DMA,
            pltpu.SemaphoreType.DMA,
        ],
    )(_agg_body)


def _make_agg_deg():
    return functools.partial(
        pl.kernel,
        mesh=plsc.VectorSubcoreMesh(**_MESH),
        out_type=[jax.ShapeDtypeStruct((NC, N_PAD, D), jnp.float32),
                  jax.ShapeDtypeStruct((NC, N_PAD, D), jnp.float32)],
        scratch_types=[
            pltpu.VMEM((2, CH), jnp.int32),
            pltpu.VMEM((2, CH), jnp.int32),
            pltpu.VMEM((TAIL,), jnp.int32),
            pltpu.VMEM((1, TAIL), jnp.int32),
            pltpu.VMEM((CH, D), jnp.float32),
            pltpu.VMEM((CH, D), jnp.float32),
            pltpu.VMEM((TAIL, D), jnp.float32),
            pltpu.VMEM_SHARED((N_PAD, D), jnp.float32),
            pltpu.SemaphoreType.DMA,
            pltpu.SemaphoreType.DMA,
            pltpu.SemaphoreType.DMA,
            pltpu.SemaphoreType.DMA,
            pltpu.SemaphoreType.DMA,
            pltpu.SemaphoreType.DMA,
            pltpu.SemaphoreType.DMA,
        ],
    )(_agg_deg_body)


def _logmap_body(x_ref, o_ref):
    v = x_ref[...]
    n = jnp.sqrt(jnp.sum(v * v, axis=1, keepdims=True))
    nc = jnp.clip(n, EPS, 1.0 - 1e-5)
    o_ref[...] = (0.5 * jnp.log((1.0 + nc) / (1.0 - nc))) * v / nc


def _tc_logmap(x):
    blk = 1000
    return pl.pallas_call(
        _logmap_body,
        out_shape=jax.ShapeDtypeStruct((N, D), jnp.float32),
        grid=(N // blk,),
        in_specs=[pl.BlockSpec((blk, D), lambda i: (i, 0))],
        out_specs=pl.BlockSpec((blk, D), lambda i: (i, 0)),
    )(x)


def _combine_body(agg_ref, deg_ref, w_ref, o_ref, *, last):
    a = agg_ref[0] + agg_ref[1]
    d = jnp.sum(deg_ref[...], axis=(0, 2)) * (1.0 / D)
    a = a / jnp.clip(d, 1.0, None)[:, None]
    out = jnp.dot(a, w_ref[...], preferred_element_type=jnp.float32)
    n = jnp.sqrt(jnp.sum(out * out, axis=1, keepdims=True))
    nc = jnp.clip(n, EPS, None)
    y = jnp.tanh(nc) * out / nc
    if not last:
        m = jnp.sqrt(jnp.sum(y * y, axis=1, keepdims=True))
        mc = jnp.clip(m, EPS, 1.0 - 1e-5)
        y = (0.5 * jnp.log((1.0 + mc) / (1.0 - mc))) * y / mc
    o_ref[...] = y


def _tc_combine(agg, deg, w, last):
    blk = 1024
    return pl.pallas_call(
        functools.partial(_combine_body, last=last),
        out_shape=jax.ShapeDtypeStruct((N_PAD, D), jnp.float32),
        grid=(N_PAD // blk,),
        in_specs=[
            pl.BlockSpec((NC, blk, D), lambda i: (0, i, 0)),
            pl.BlockSpec((NC, blk, D), lambda i: (0, i, 0)),
            pl.BlockSpec((D, D), lambda i: (0, 0)),
        ],
        out_specs=pl.BlockSpec((blk, D), lambda i: (i, 0)),
    )(agg, deg, w)


def kernel(x, edge_index, W0, W1):
    src = edge_index[0].astype(jnp.int32)
    dst = edge_index[1].astype(jnp.int32)
    h0 = _tc_logmap(x)
    agg1, deg = _make_agg_deg()(h0, src, dst)
    h1 = _tc_combine(agg1, deg, W0, last=False)
    agg2 = _make_agg()(h1, src, dst)
    y = _tc_combine(agg2, deg, W1, last=True)
    return y[:N]
